# Initial kernel scaffold; baseline (speedup 1.0000x reference)
import jax
import jax.numpy as jnp
from jax.experimental import pallas as pl


def kernel(x, router_prob, alpha, ln_w, ln_b, l1_w, l1_b, l2_w, l2_b):
    raise NotImplementedError("write your pallas kernel here")


# speedup vs baseline: 1.9992x; 1.9992x over previous
import functools

import jax
import jax.numpy as jnp
from jax.experimental import pallas as pl
from jax.experimental.pallas import tpu as pltpu

MODEL_DIM = 768
NUM_EXPERTS = 8
INNER = MODEL_DIM * 4
BLK = 256


def _mlp_body(alpha_ref, ei_ref, sp_ref, w1_ref, b1_ref, w2_ref, b2_ref,
              lnw_ref, lnb_ref, out_ref, *, e, m, mp):
    ei = ei_ref[...]
    mu = jnp.mean(ei, axis=1, keepdims=True)
    var = jnp.mean((ei - mu) ** 2, axis=1, keepdims=True)
    h = (ei - mu) / jnp.sqrt(var + 1e-5) * lnw_ref[...] + lnb_ref[...]
    ext = h[:, :mp]
    if m < mp:
        col = jax.lax.broadcasted_iota(jnp.int32, (1, mp), 1)
        ext = jnp.where(col < m, ext, 0.0)
    a = jax.lax.dot_general(ext, w1_ref[...],
                            dimension_numbers=(((1,), (1,)), ((), ())),
                            preferred_element_type=jnp.float32) + b1_ref[...]
    inner = 0.5 * a * (1.0 + jax.lax.erf(a * 0.7071067811865476))
    outp = jax.lax.dot_general(inner, w2_ref[...],
                               dimension_numbers=(((1,), (1,)), ((), ())),
                               preferred_element_type=jnp.float32) + b2_ref[...]
    if m < mp:
        outp = jnp.where(col < m, outp, 0.0)
    scale = alpha_ref[0, 0] * sp_ref[:, e:e + 1] + 1.0
    if mp == MODEL_DIM:
        out_ref[...] = ei + scale * outp
    else:
        out_ref[...] = jnp.concatenate(
            [ei[:, :mp] + scale * outp, ei[:, mp:]], axis=1)


def _expert_mlp(e, ei, sp, alpha2d, l1_w, l1_b2d, l2_w, l2_b2d, lnw2d, lnb2d):
    m = MODEL_DIM >> e
    mp = max(128, ((m + 127) // 128) * 128)
    rows = ei.shape[0]
    grid = (rows // BLK,)
    return pl.pallas_call(
        functools.partial(_mlp_body, e=e, m=m, mp=mp),
        grid=grid,
        in_specs=[
            pl.BlockSpec(memory_space=pltpu.SMEM),
            pl.BlockSpec((BLK, MODEL_DIM), lambda i: (i, 0)),
            pl.BlockSpec((BLK, NUM_EXPERTS), lambda i: (i, 0)),
            pl.BlockSpec((INNER, mp), lambda i: (0, 0)),
            pl.BlockSpec((1, INNER), lambda i: (0, 0)),
            pl.BlockSpec((mp, INNER), lambda i: (0, 0)),
            pl.BlockSpec((1, mp), lambda i: (0, 0)),
            pl.BlockSpec((1, MODEL_DIM), lambda i: (0, 0)),
            pl.BlockSpec((1, MODEL_DIM), lambda i: (0, 0)),
        ],
        out_specs=pl.BlockSpec((BLK, MODEL_DIM), lambda i: (i, 0)),
        out_shape=jax.ShapeDtypeStruct((rows, MODEL_DIM), jnp.float32),
    )(alpha2d, ei, sp, l1_w, l1_b2d, l2_w, l2_b2d, lnw2d, lnb2d)


def _route(raw_rp, x):
    Bb, Tt, E = raw_rp.shape
    n = Tt // E
    rp = raw_rp
    eis, sps = [], []
    brow = jnp.arange(Bb)[:, None]
    for e in range(E):
        _, idx = jax.lax.top_k(rp[:, :, e], n)
        eis.append(jnp.take_along_axis(x, idx[:, :, None], axis=1))
        sps.append(jnp.take_along_axis(raw_rp, idx[:, :, None], axis=1))
        rp = rp.at[brow, idx, :].set(0.0)
    return eis, sps


def kernel(x, router_prob, alpha, ln_w, ln_b, l1_w, l1_b, l2_w, l2_b):
    Bb, Tt, d = x.shape
    eis, sps = _route(router_prob, x)
    alpha2d = jnp.reshape(alpha, (1, 1)).astype(jnp.float32)
    l1_b2d = l1_b.reshape(1, INNER)
    l2_b2d = l2_b.reshape(1, MODEL_DIM)
    lnw2d = ln_w.reshape(1, MODEL_DIM)
    lnb2d = ln_b.reshape(1, MODEL_DIM)
    outs = []
    for e in range(NUM_EXPERTS):
        ei = eis[e].reshape(-1, d)
        sp = sps[e].reshape(-1, NUM_EXPERTS)
        o = _expert_mlp(e, ei, sp, alpha2d, l1_w, l1_b2d, l2_w, l2_b2d,
                        lnw2d, lnb2d)
        outs.append(o.reshape(Bb, -1, d))
    return jnp.concatenate(outs, axis=1), jnp.concatenate(sps, axis=1)
